# baseline (speedup 1.0000x reference)
import functools
import jax
import jax.numpy as jnp
from jax import lax
from jax.experimental import pallas as pl
from jax.experimental.pallas import tpu as pltpu
from jax.experimental.pallas import tpu_sc as plsc

_D = 32
_CHUNK = 208


def _make_sc_gather(n_idx):
    info = plsc.get_sparse_core_info()
    nw = info.num_cores * info.num_subcores
    per_w = n_idx // nw
    assert per_w * nw == n_idx and per_w % _CHUNK == 0
    n_ch = per_w // _CHUNK
    assert n_ch % 2 == 0
    mesh = plsc.VectorSubcoreMesh(core_axis_name="c", subcore_axis_name="s")

    @functools.partial(
        pl.kernel,
        out_type=jax.ShapeDtypeStruct((n_idx, _D), jnp.float32),
        mesh=mesh,
        scratch_types=[
            pltpu.VMEM((_CHUNK,), jnp.int32),
            pltpu.VMEM((_CHUNK,), jnp.int32),
            pltpu.VMEM((_CHUNK,), jnp.int32),
            pltpu.VMEM((_CHUNK,), jnp.int32),
            pltpu.VMEM((_CHUNK,), jnp.int32),
            pltpu.VMEM((_CHUNK, 128), jnp.float32),
            pltpu.VMEM((_CHUNK, 128), jnp.float32),
            pltpu.VMEM((_CHUNK, _D), jnp.float32),
            pltpu.SemaphoreType.DMA,
            pltpu.SemaphoreType.DMA,
        ],
        compiler_params=pltpu.CompilerParams(
            use_tc_tiling_on_sc=True, needs_layout_passes=False
        ),
    )
    def gather(t_hbm, idx_hbm, out_hbm, itmp_v, q0_v, q1_v, r0_v, r1_v,
               g0_v, g1_v, obuf_v, sem0, sem1):
        wid = lax.axis_index("s") * info.num_cores + lax.axis_index("c")
        base = wid * per_w
        lanes = lax.iota(jnp.int32, 16)
        qv = [q0_v, q1_v]
        rv = [r0_v, r1_v]
        gv = [g0_v, g1_v]
        sems = [sem0, sem1]
        ng = _CHUNK // 16

        def prep(c, par):
            pltpu.sync_copy(idx_hbm.at[pl.ds(base + c * _CHUNK, _CHUNK)], itmp_v)
            for k in range(ng):
                v16 = itmp_v[pl.ds(k * 16, 16)]
                qv[par][pl.ds(k * 16, 16)] = lax.shift_right_logical(v16, 2)
                rv[par][pl.ds(k * 16, 16)] = (v16 & 3) * _D

        def issue(par):
            pltpu.async_copy(t_hbm.at[qv[par]], gv[par], sems[par])

        def drain(par):
            pltpu.make_async_copy(t_hbm.at[qv[par]], gv[par], sems[par]).wait()

        def extract_flush(c, par):
            for k in range(ng):
                k16 = k * 16 + lanes
                r16 = rv[par][pl.ds(k * 16, 16)]
                for dd in range(_D):
                    vals = plsc.load_gather(gv[par], [k16, r16 + dd])
                    plsc.store_scatter(
                        obuf_v, [k16, jnp.full((16,), dd, jnp.int32)], vals
                    )
            pltpu.sync_copy(obuf_v, out_hbm.at[pl.ds(base + c * _CHUNK, _CHUNK)])

        prep(jnp.int32(0), 0)
        issue(0)

        def body(s, _):
            a = 2 * s

            prep(a + 1, 1)
            issue(1)
            drain(0)
            extract_flush(a, 0)

            @pl.when(s + 1 < n_ch // 2)
            def _():
                prep(a + 2, 0)
                issue(0)

            drain(1)
            extract_flush(a + 1, 1)
            return 0

        lax.fori_loop(0, n_ch // 2, body, 0)

    return gather


def _plr_body(x_ref, e_ref, bins_ref, wt_ref, b_ref, o_ref):
    xb = jnp.dot(x_ref[...], e_ref[...], preferred_element_type=jnp.float32)
    plr = jnp.maximum(1.0 - jnp.abs(xb - bins_ref[...]), 0.0)
    o_ref[...] = (
        jnp.dot(plr, wt_ref[...], preferred_element_type=jnp.float32) + b_ref[...]
    )


def kernel(x_cat, x_cont, tables, bin_boundaries, W, b, cls_token):
    bsz = x_cat.shape[0]
    nf, vocab, d = tables.shape
    ncf, nbins = bin_boundaries.shape

    t128 = tables.reshape(nf * vocab * d // 128, 128)
    offs = (jnp.arange(nf, dtype=jnp.int32) * vocab)[None, :]
    idx_flat = (x_cat.astype(jnp.int32) + offs).reshape(-1)
    gather = _make_sc_gather(bsz * nf)
    cat_t = gather(t128, idx_flat).reshape(bsz, nf, d)

    expand = jnp.repeat(jnp.eye(ncf, dtype=jnp.float32), nbins, axis=1)
    bins_row = bin_boundaries.reshape(1, ncf * nbins)
    b_row = b.reshape(1, ncf * d)
    blk = 512
    cont_flat = pl.pallas_call(
        _plr_body,
        out_shape=jax.ShapeDtypeStruct((bsz, ncf * d), jnp.float32),
        grid=(bsz // blk,),
        in_specs=[
            pl.BlockSpec((blk, ncf), lambda i: (i, 0)),
            pl.BlockSpec((ncf, ncf * nbins), lambda i: (0, 0)),
            pl.BlockSpec((1, ncf * nbins), lambda i: (0, 0)),
            pl.BlockSpec((ncf * nbins, ncf * d), lambda i: (0, 0)),
            pl.BlockSpec((1, ncf * d), lambda i: (0, 0)),
        ],
        out_specs=pl.BlockSpec((blk, ncf * d), lambda i: (i, 0)),
    )(x_cont, expand, bins_row, W.T, b_row)
    cont_t = cont_flat.reshape(bsz, ncf, d)

    cls_t = jnp.broadcast_to(cls_token, (bsz, 1, d))
    return jnp.concatenate([cls_t, cat_t, cont_t], axis=1)


# speedup vs baseline: 1.1256x; 1.1256x over previous
import functools
import jax
import jax.numpy as jnp
from jax import lax
from jax.experimental import pallas as pl
from jax.experimental.pallas import tpu as pltpu
from jax.experimental.pallas import tpu_sc as plsc

_NF = 26
_VOCAB = 100000
_D = 32
_NC_FIELDS = 13
_NBINS = 16


def _make_sc_gather(n_rows, d, n_idx):
    info = plsc.get_sparse_core_info()
    ncores, nsub = info.num_cores, info.num_subcores
    nw = ncores * nsub
    assert n_idx % nw == 0
    per_w = n_idx // nw
    assert (per_w * 8) % 8 == 0

    mesh = plsc.VectorSubcoreMesh(core_axis_name="c", subcore_axis_name="s")

    @functools.partial(
        pl.kernel,
        out_type=jax.ShapeDtypeStruct((n_idx, d), jnp.float32),
        mesh=mesh,
        scratch_types=[
            pltpu.VMEM((per_w,), jnp.int32),
            pltpu.VMEM((per_w, d), jnp.float32),
            pltpu.SemaphoreType.DMA,
        ],
        compiler_params=pltpu.CompilerParams(use_tc_tiling_on_sc=False),
    )
    def gather(table_hbm, idx_hbm, out_hbm, idx_v, rows_v, sem):
        wid = lax.axis_index("s") * ncores + lax.axis_index("c")
        base = wid * per_w
        pltpu.sync_copy(idx_hbm.at[pl.ds(base, per_w)], idx_v)
        pltpu.async_copy(table_hbm.at[idx_v], rows_v, sem).wait()
        pltpu.sync_copy(rows_v, out_hbm.at[pl.ds(base, per_w)])

    return gather


def _plr_body(x_ref, e_ref, bins_ref, wt_ref, b_ref, o_ref):
    xb = jnp.dot(x_ref[...], e_ref[...], preferred_element_type=jnp.float32)
    plr = jnp.maximum(1.0 - jnp.abs(xb - bins_ref[...]), 0.0)
    o_ref[...] = (
        jnp.dot(plr, wt_ref[...], preferred_element_type=jnp.float32) + b_ref[...]
    )


def kernel(x_cat, x_cont, tables, bin_boundaries, W, b, cls_token):
    bsz = x_cat.shape[0]
    nf, vocab, d = tables.shape
    ncf, nbins = bin_boundaries.shape

    offs = (jnp.arange(nf, dtype=jnp.int32) * vocab)[None, :]
    idx_flat = (x_cat.astype(jnp.int32) + offs).reshape(-1)
    table_flat = tables.reshape(nf * vocab, d)
    gather = _make_sc_gather(nf * vocab, d, bsz * nf)
    cat_t = gather(table_flat, idx_flat).reshape(bsz, nf, d)

    expand = (
        jnp.repeat(jnp.eye(ncf, dtype=jnp.float32), nbins, axis=1)
    )
    bins_row = bin_boundaries.reshape(1, ncf * nbins)
    b_row = b.reshape(1, ncf * d)
    blk = 512
    grid = (bsz // blk,)
    cont_flat = pl.pallas_call(
        _plr_body,
        out_shape=jax.ShapeDtypeStruct((bsz, ncf * d), jnp.float32),
        grid=grid,
        in_specs=[
            pl.BlockSpec((blk, ncf), lambda i: (i, 0)),
            pl.BlockSpec((ncf, ncf * nbins), lambda i: (0, 0)),
            pl.BlockSpec((1, ncf * nbins), lambda i: (0, 0)),
            pl.BlockSpec((ncf * nbins, ncf * d), lambda i: (0, 0)),
            pl.BlockSpec((1, ncf * d), lambda i: (0, 0)),
        ],
        out_specs=pl.BlockSpec((blk, ncf * d), lambda i: (i, 0)),
    )(x_cont, expand, bins_row, W.T, b_row)
    cont_t = cont_flat.reshape(bsz, ncf, d)

    cls_t = jnp.broadcast_to(cls_token, (bsz, 1, d))
    return jnp.concatenate([cls_t, cat_t, cont_t], axis=1)
